# baseline (speedup 1.0000x reference)
import functools

import jax
import jax.numpy as jnp
from jax import lax
from jax.experimental import pallas as pl
from jax.experimental.pallas import tpu as pltpu
from jax.experimental.pallas import tpu_sc as plsc

TEMPERATURE = 2.0
FACTOR = 0.8
NEG = 256
EPS = 1e-08

B = 4
C = 192
H = 24
W = 24
HW = H * W


def _prep_body(z_ref, v_ref, rgb_ref, ni_ref, a_ref, j_ref, s0_ref, w_scr):
    b = pl.program_id(0)

    @pl.when(b == 0)
    def _():
        pi = lax.broadcasted_iota(jnp.int32, (HW, HW), 0)
        qi = lax.broadcasted_iota(jnp.int32, (HW, HW), 1)
        dr = ((pi // W) - (qi // W)).astype(jnp.float32)
        dc = ((pi % W) - (qi % W)).astype(jnp.float32)
        diag = float(((H - 1) ** 2 + (W - 1) ** 2) ** 0.5)
        deuc = jnp.sqrt(dr * dr + dc * dc) * (1.0 / diag)
        acc = jnp.zeros((HW, HW), jnp.float32)
        for k in range(3):
            rk = rgb_ref[k]
            rp = lax.broadcast_in_dim(rk, (HW, HW), (0,))
            rq = lax.broadcast_in_dim(rk, (HW, HW), (1,))
            acc = acc + (rp - rq) * (rp - rq)
        drgb = jnp.sqrt(acc) * (1.0 / (3.0 ** 0.5))
        w_scr[...] = deuc * FACTOR + drgb * (1.0 - FACTOR)

    z = z_ref[0].reshape(C, HW)
    v = v_ref[0].reshape(C, HW)
    g = lax.dot_general(z, v, (((0,), (0,)), ((), ())),
                        preferred_element_type=jnp.float32)
    n1sq = jnp.sum(z * z, axis=0)
    n2sq = jnp.sum(v * v, axis=0)
    n1m = lax.broadcast_in_dim(jnp.sqrt(n1sq), (HW, HW), (0,))
    n2m = lax.broadcast_in_dim(jnp.sqrt(n2sq), (HW, HW), (1,))
    den = jnp.maximum(n1m * n2m, EPS)
    a_ref[0] = jnp.minimum(jnp.abs(g) * w_scr[...] / den, 1.0)
    j_ref[0] = ni_ref[0, 0] * W + ni_ref[0, 1]
    s0pix = jnp.minimum(n1sq / jnp.maximum(n1sq, EPS), 1.0)
    s0_ref[...] = jnp.full((1, 1, 128), jnp.sum(s0pix) * (1.0 / HW),
                           jnp.float32)


def _finish_body(p_ref, s0_ref, o_ref):
    parts = p_ref[...]
    negsum = parts[:, 0, :]
    for wkr in range(1, parts.shape[1]):
        negsum = negsum + parts[:, wkr, :]
    sim = negsum * (1.0 / (HW * TEMPERATURE))
    s0 = s0_ref[...][:, 0, 0:1]
    logp0 = jnp.clip(jnp.log(s0), -100.0, None)
    log1m = jnp.clip(jnp.log(1.0 - sim), -100.0, None)
    bce = -(logp0 + jnp.sum(log1m, axis=1, keepdims=True)) * (1.0 / (NEG + 1))
    loss = jnp.sum(bce) * (1.0 / B)
    out2 = jnp.sum(s0) * (1.0 / B)
    out3 = jnp.sum(sim) * (TEMPERATURE / (NEG * B))
    lanes = lax.broadcasted_iota(jnp.int32, (8, 128), 1)
    res = jnp.where(lanes == 0, loss,
                    jnp.where(lanes == 1, out2,
                              jnp.where(lanes == 2, out3, 0.0)))
    o_ref[...] = res


def _make_sc_gather(nc, nw, ppw):
    mesh = plsc.VectorSubcoreMesh(core_axis_name="c", subcore_axis_name="s")

    @functools.partial(
        pl.kernel,
        out_type=jax.ShapeDtypeStruct((B, nw, NEG), jnp.float32),
        mesh=mesh,
        scratch_types=[
            pltpu.VMEM((ppw, HW), jnp.float32),
            pltpu.VMEM((ppw, NEG), jnp.int32),
            pltpu.VMEM((NEG,), jnp.float32),
        ],
        compiler_params=pltpu.CompilerParams(use_tc_tiling_on_sc=False,
                                             needs_layout_passes=False),
    )
    def sc_gather(a_hbm, j_hbm, out_hbm, rows_v, idx_v, acc_v):
        cid = lax.axis_index("c")
        sid = lax.axis_index("s")
        wid = sid * nc + cid
        base = wid * ppw
        lane = lax.iota(jnp.int32, 16)
        for b in range(B):
            pltpu.sync_copy(a_hbm.at[b, pl.ds(base, ppw)], rows_v)
            pltpu.sync_copy(j_hbm.at[b, pl.ds(base, ppw)], idx_v)

            def tbody(t, accs):
                tv = jnp.full((16,), t, jnp.int32)
                new = []
                for i in range(NEG // 16):
                    col = lane + (i * 16)
                    jv = plsc.load_gather(idx_v, [tv, col])
                    gv = plsc.load_gather(rows_v, [tv, jv])
                    new.append(accs[i] + gv)
                return tuple(new)

            accs = lax.fori_loop(
                0, ppw, tbody,
                tuple(jnp.zeros((16,), jnp.float32) for _ in range(NEG // 16)))
            for i in range(NEG // 16):
                acc_v[pl.ds(i * 16, 16)] = accs[i]
            pltpu.sync_copy(acc_v, out_hbm.at[b, wid])

    return sc_gather


def _run_prep(z1, v2, rgb, ni, interpret=False):
    return pl.pallas_call(
        _prep_body,
        grid=(B,),
        in_specs=[
            pl.BlockSpec((1, C, H, W), lambda b: (b, 0, 0, 0)),
            pl.BlockSpec((1, C, H, W), lambda b: (b, 0, 0, 0)),
            pl.BlockSpec((3, HW), lambda b: (0, 0)),
            pl.BlockSpec((1, 2, HW, NEG), lambda b: (b, 0, 0, 0)),
        ],
        out_specs=[
            pl.BlockSpec((1, HW, HW), lambda b: (b, 0, 0)),
            pl.BlockSpec((1, HW, NEG), lambda b: (b, 0, 0)),
            pl.BlockSpec((1, 1, 128), lambda b: (b, 0, 0)),
        ],
        out_shape=[
            jax.ShapeDtypeStruct((B, HW, HW), jnp.float32),
            jax.ShapeDtypeStruct((B, HW, NEG), jnp.int32),
            jax.ShapeDtypeStruct((B, 1, 128), jnp.float32),
        ],
        scratch_shapes=[pltpu.VMEM((HW, HW), jnp.float32)],
        interpret=interpret,
    )(z1, v2, rgb, ni)


def _run_finish(partials, s0, nw, interpret=False):
    return pl.pallas_call(
        _finish_body,
        in_specs=[
            pl.BlockSpec((B, nw, NEG), lambda: (0, 0, 0)),
            pl.BlockSpec((B, 1, 128), lambda: (0, 0, 0)),
        ],
        out_specs=pl.BlockSpec((8, 128), lambda: (0, 0)),
        out_shape=jax.ShapeDtypeStruct((8, 128), jnp.float32),
        interpret=interpret,
    )(partials, s0)


@jax.jit
def kernel(views_1, views_2, img, neg_idx):
    rgb = img[0].reshape(3, HW)
    a_mat, jflat, s0 = _run_prep(views_1, views_2, rgb, neg_idx)

    info = plsc.get_sparse_core_info()
    nw = info.num_cores * info.num_subcores
    ppw = HW // nw
    partials = _make_sc_gather(info.num_cores, nw, ppw)(a_mat, jflat)

    res = _run_finish(partials, s0, nw)
    return res[0, 0], res[0, 1], res[0, 2]


# speedup vs baseline: 1.2974x; 1.2974x over previous
import functools

import jax
import jax.numpy as jnp
from jax import lax
from jax.experimental import pallas as pl
from jax.experimental.pallas import tpu as pltpu
from jax.experimental.pallas import tpu_sc as plsc

TEMPERATURE = 2.0
FACTOR = 0.8
NEG = 256
EPS = 1e-08

B = 4
C = 192
H = 24
W = 24
HW = H * W


def _prep_body(z_ref, v_ref, rgb_ref, ni_ref, a_ref, j_ref, s0_ref, w_scr):
    b = pl.program_id(0)

    @pl.when(b == 0)
    def _():
        p1 = lax.iota(jnp.int32, HW)
        rows = (p1 // W).astype(jnp.float32)
        cols = (p1 % W).astype(jnp.float32)
        dr = (lax.broadcast_in_dim(rows, (HW, HW), (0,))
              - lax.broadcast_in_dim(rows, (HW, HW), (1,)))
        dc = (lax.broadcast_in_dim(cols, (HW, HW), (0,))
              - lax.broadcast_in_dim(cols, (HW, HW), (1,)))
        diag = float(((H - 1) ** 2 + (W - 1) ** 2) ** 0.5)
        deuc = jnp.sqrt(dr * dr + dc * dc) * (1.0 / diag)
        acc = jnp.zeros((HW, HW), jnp.float32)
        for k in range(3):
            rk = rgb_ref[k]
            rp = lax.broadcast_in_dim(rk, (HW, HW), (0,))
            rq = lax.broadcast_in_dim(rk, (HW, HW), (1,))
            acc = acc + (rp - rq) * (rp - rq)
        drgb = jnp.sqrt(acc) * (1.0 / (3.0 ** 0.5))
        w_scr[...] = deuc * FACTOR + drgb * (1.0 - FACTOR)

    z = z_ref[0]
    v = v_ref[0]
    g = lax.dot_general(z, v, (((0,), (0,)), ((), ())),
                        preferred_element_type=jnp.float32)
    n1sq = jnp.sum(z * z, axis=0)
    n2sq = jnp.sum(v * v, axis=0)
    n1 = jnp.sqrt(n1sq)
    n2 = jnp.sqrt(n2sq)
    n1m = lax.broadcast_in_dim(n1, (HW, HW), (0,))
    n2m = lax.broadcast_in_dim(n2, (HW, HW), (1,))
    r1m = lax.broadcast_in_dim(1.0 / n1, (HW, HW), (0,))
    r2m = lax.broadcast_in_dim(1.0 / n2, (HW, HW), (1,))
    t = jnp.abs(g) * w_scr[...]
    a = jnp.where(n1m * n2m < EPS, t * (1.0 / EPS), t * r1m * r2m)
    a_ref[0] = jnp.minimum(a, 1.0)
    j_ref[0] = ni_ref[0, 0] * W + ni_ref[0, 1]
    s0pix = jnp.minimum(n1sq / jnp.maximum(n1sq, EPS), 1.0)
    s0_ref[...] = jnp.full((1, 1, 128), jnp.sum(s0pix) * (1.0 / HW),
                           jnp.float32)


def _finish_body(p_ref, s0_ref, o_ref):
    parts = p_ref[...]
    negsum = parts[:, 0, :]
    for wkr in range(1, parts.shape[1]):
        negsum = negsum + parts[:, wkr, :]
    sim = negsum * (1.0 / (HW * TEMPERATURE))
    s0 = s0_ref[...][:, 0, 0:1]
    logp0 = jnp.clip(jnp.log(s0), -100.0, None)
    log1m = jnp.clip(jnp.log(1.0 - sim), -100.0, None)
    bce = -(logp0 + jnp.sum(log1m, axis=1, keepdims=True)) * (1.0 / (NEG + 1))
    loss = jnp.sum(bce) * (1.0 / B)
    out2 = jnp.sum(s0) * (1.0 / B)
    out3 = jnp.sum(sim) * (TEMPERATURE / (NEG * B))
    lanes = lax.broadcasted_iota(jnp.int32, (8, 128), 1)
    res = jnp.where(lanes == 0, loss,
                    jnp.where(lanes == 1, out2,
                              jnp.where(lanes == 2, out3, 0.0)))
    o_ref[...] = res


def _make_sc_gather(nc, nw, ppw):
    mesh = plsc.VectorSubcoreMesh(core_axis_name="c", subcore_axis_name="s")

    @functools.partial(
        pl.kernel,
        out_type=jax.ShapeDtypeStruct((B, nw, NEG), jnp.float32),
        mesh=mesh,
        scratch_types=[
            pltpu.VMEM((2, ppw, HW), jnp.float32),
            pltpu.VMEM((2, ppw, NEG), jnp.int32),
            pltpu.VMEM((NEG,), jnp.float32),
            pltpu.SemaphoreType.DMA,
        ],
        compiler_params=pltpu.CompilerParams(use_tc_tiling_on_sc=False,
                                             needs_layout_passes=False),
    )
    def sc_gather(a_hbm, j_hbm, out_hbm, rows_v, idx_v, acc_v, sem):
        cid = lax.axis_index("c")
        sid = lax.axis_index("s")
        wid = sid * nc + cid
        base = wid * ppw
        lane = lax.iota(jnp.int32, 16)

        def start(b, k):
            ca = pltpu.async_copy(a_hbm.at[b, pl.ds(base, ppw)],
                                  rows_v.at[k], sem)
            cj = pltpu.async_copy(j_hbm.at[b, pl.ds(base, ppw)],
                                  idx_v.at[k], sem)
            return ca, cj

        pend = start(0, 0)
        for b in range(B):
            k = b % 2
            kv = jnp.full((16,), k, jnp.int32)
            pend[0].wait()
            pend[1].wait()
            if b + 1 < B:
                pend = start(b + 1, 1 - k)

            def tbody(t, accs):
                tv = jnp.full((16,), t, jnp.int32)
                new = []
                for i in range(NEG // 16):
                    col = lane + (i * 16)
                    jv = plsc.load_gather(idx_v, [kv, tv, col])
                    gv = plsc.load_gather(rows_v, [kv, tv, jv])
                    new.append(accs[i] + gv)
                return tuple(new)

            accs = lax.fori_loop(
                0, ppw, tbody,
                tuple(jnp.zeros((16,), jnp.float32) for _ in range(NEG // 16)))
            for i in range(NEG // 16):
                acc_v[pl.ds(i * 16, 16)] = accs[i]
            pltpu.sync_copy(acc_v, out_hbm.at[b, wid])

    return sc_gather


def _run_prep(z1, v2, rgb, ni, interpret=False):
    return pl.pallas_call(
        _prep_body,
        grid=(B,),
        in_specs=[
            pl.BlockSpec((1, C, HW), lambda b: (b, 0, 0)),
            pl.BlockSpec((1, C, HW), lambda b: (b, 0, 0)),
            pl.BlockSpec((3, HW), lambda b: (0, 0)),
            pl.BlockSpec((1, 2, HW, NEG), lambda b: (b, 0, 0, 0)),
        ],
        out_specs=[
            pl.BlockSpec((1, HW, HW), lambda b: (b, 0, 0)),
            pl.BlockSpec((1, HW, NEG), lambda b: (b, 0, 0)),
            pl.BlockSpec((1, 1, 128), lambda b: (b, 0, 0)),
        ],
        out_shape=[
            jax.ShapeDtypeStruct((B, HW, HW), jnp.float32),
            jax.ShapeDtypeStruct((B, HW, NEG), jnp.int32),
            jax.ShapeDtypeStruct((B, 1, 128), jnp.float32),
        ],
        scratch_shapes=[pltpu.VMEM((HW, HW), jnp.float32)],
        interpret=interpret,
    )(z1, v2, rgb, ni)


def _run_finish(partials, s0, nw, interpret=False):
    return pl.pallas_call(
        _finish_body,
        in_specs=[
            pl.BlockSpec((B, nw, NEG), lambda: (0, 0, 0)),
            pl.BlockSpec((B, 1, 128), lambda: (0, 0, 0)),
        ],
        out_specs=pl.BlockSpec((8, 128), lambda: (0, 0)),
        out_shape=jax.ShapeDtypeStruct((8, 128), jnp.float32),
        interpret=interpret,
    )(partials, s0)


@jax.jit
def kernel(views_1, views_2, img, neg_idx):
    z1 = views_1.reshape(B, C, HW)
    v2 = views_2.reshape(B, C, HW)
    rgb = img[0].reshape(3, HW)
    a_mat, jflat, s0 = _run_prep(z1, v2, rgb, neg_idx)

    info = plsc.get_sparse_core_info()
    nw = info.num_cores * info.num_subcores
    ppw = HW // nw
    partials = _make_sc_gather(info.num_cores, nw, ppw)(a_mat, jflat)

    res = _run_finish(partials, s0, nw)
    return res[0, 0], res[0, 1], res[0, 2]


# speedup vs baseline: 1.6346x; 1.2599x over previous
import functools

import jax
import jax.numpy as jnp
from jax import lax
from jax.experimental import pallas as pl
from jax.experimental.pallas import tpu as pltpu
from jax.experimental.pallas import tpu_sc as plsc

TEMPERATURE = 2.0
FACTOR = 0.8
NEG = 256
EPS = 1e-08

B = 4
C = 192
H = 24
W = 24
HW = H * W
NQ = 5
NJ = NEG // 128


def _prep_body(z_ref, v_ref, rgb_ref, ni_ref, a_ref, j_ref, s0_ref, w_scr):
    b = pl.program_id(0)

    @pl.when(b == 0)
    def _():
        p1 = lax.iota(jnp.int32, HW)
        rows = (p1 // W).astype(jnp.float32)
        cols = (p1 % W).astype(jnp.float32)
        dr = (lax.broadcast_in_dim(rows, (HW, HW), (0,))
              - lax.broadcast_in_dim(rows, (HW, HW), (1,)))
        dc = (lax.broadcast_in_dim(cols, (HW, HW), (0,))
              - lax.broadcast_in_dim(cols, (HW, HW), (1,)))
        diag = float(((H - 1) ** 2 + (W - 1) ** 2) ** 0.5)
        deuc = jnp.sqrt(dr * dr + dc * dc) * (1.0 / diag)
        acc = jnp.zeros((HW, HW), jnp.float32)
        for k in range(3):
            rk = rgb_ref[k]
            rp = lax.broadcast_in_dim(rk, (HW, HW), (0,))
            rq = lax.broadcast_in_dim(rk, (HW, HW), (1,))
            acc = acc + (rp - rq) * (rp - rq)
        drgb = jnp.sqrt(acc) * (1.0 / (3.0 ** 0.5))
        w_scr[...] = deuc * FACTOR + drgb * (1.0 - FACTOR)

    z = z_ref[0]
    v = v_ref[0]
    g = lax.dot_general(z, v, (((0,), (0,)), ((), ())),
                        preferred_element_type=jnp.float32)
    n1sq = jnp.sum(z * z, axis=0)
    n2sq = jnp.sum(v * v, axis=0)
    n1 = jnp.sqrt(n1sq)
    n2 = jnp.sqrt(n2sq)
    n1m = lax.broadcast_in_dim(n1, (HW, HW), (0,))
    n2m = lax.broadcast_in_dim(n2, (HW, HW), (1,))
    r1m = lax.broadcast_in_dim(1.0 / n1, (HW, HW), (0,))
    r2m = lax.broadcast_in_dim(1.0 / n2, (HW, HW), (1,))
    t = jnp.abs(g) * w_scr[...]
    a = jnp.where(n1m * n2m < EPS, t * (1.0 / EPS), t * r1m * r2m)
    a = jnp.minimum(a, 1.0)
    for qt in range(NQ):
        wdt = min(128, HW - qt * 128)
        a_ref[0, pl.ds(qt * HW, HW), pl.ds(0, wdt)] = lax.slice(
            a, (0, qt * 128), (HW, qt * 128 + wdt))
    jf = ni_ref[0, 0] * W + ni_ref[0, 1]
    for nt in range(NJ):
        j_ref[0, pl.ds(nt * HW, HW), :] = lax.slice(
            jf, (0, nt * 128), (HW, nt * 128 + 128))
    s0pix = jnp.minimum(n1sq / jnp.maximum(n1sq, EPS), 1.0)
    s0_ref[...] = jnp.full((1, 1, 128), jnp.sum(s0pix) * (1.0 / HW),
                           jnp.float32)


def _finish_body(p_ref, s0_ref, o_ref):
    parts = p_ref[...]
    ns = [jnp.zeros((B, 128), jnp.float32) for _ in range(NJ)]
    for wkr in range(parts.shape[1] // NJ):
        for nt in range(NJ):
            ns[nt] = ns[nt] + parts[:, NJ * wkr + nt, :]
    sims = [x * (1.0 / (HW * TEMPERATURE)) for x in ns]
    s0 = s0_ref[...][:, 0, 0:1]
    logp0 = jnp.clip(jnp.log(s0), -100.0, None)
    l1m = sum(jnp.sum(jnp.clip(jnp.log(1.0 - s), -100.0, None),
                      axis=1, keepdims=True) for s in sims)
    bce = -(logp0 + l1m) * (1.0 / (NEG + 1))
    loss = jnp.sum(bce) * (1.0 / B)
    out2 = jnp.sum(s0) * (1.0 / B)
    out3 = sum(jnp.sum(s) for s in sims) * (TEMPERATURE / (NEG * B))
    lanes = lax.broadcasted_iota(jnp.int32, (8, 128), 1)
    res = jnp.where(lanes == 0, loss,
                    jnp.where(lanes == 1, out2,
                              jnp.where(lanes == 2, out3, 0.0)))
    o_ref[...] = res


def _make_sc_gather(nc, nw, ppw):
    mesh = plsc.VectorSubcoreMesh(core_axis_name="c", subcore_axis_name="s")

    @functools.partial(
        pl.kernel,
        out_type=jax.ShapeDtypeStruct((B, NJ * nw, 128), jnp.float32),
        mesh=mesh,
        scratch_types=[
            pltpu.VMEM((2, NQ, ppw, 128), jnp.float32),
            pltpu.VMEM((2, NJ, ppw, 128), jnp.int32),
            pltpu.VMEM((NJ, 128), jnp.float32),
            pltpu.SemaphoreType.DMA,
        ],
        compiler_params=pltpu.CompilerParams(use_tc_tiling_on_sc=False,
                                             needs_layout_passes=False),
    )
    def sc_gather(a_hbm, j_hbm, out_hbm, rows_v, idx_v, acc_v, sem):
        cid = lax.axis_index("c")
        sid = lax.axis_index("s")
        wid = sid * nc + cid
        base = wid * ppw
        lane = lax.iota(jnp.int32, 16)

        def start(b, k):
            hs = []
            for qt in range(NQ):
                hs.append(pltpu.async_copy(
                    a_hbm.at[b, pl.ds(qt * HW + base, ppw)],
                    rows_v.at[k, qt], sem))
            for nt in range(NJ):
                hs.append(pltpu.async_copy(
                    j_hbm.at[b, pl.ds(nt * HW + base, ppw)],
                    idx_v.at[k, nt], sem))
            return hs

        pend = start(0, 0)
        for b in range(B):
            k = b % 2
            kv = jnp.full((16,), k, jnp.int32)
            for h in pend:
                h.wait()
            if b + 1 < B:
                pend = start(b + 1, 1 - k)

            def tbody(t, accs):
                tv = jnp.full((16,), t, jnp.int32)
                new = []
                for i in range(NEG // 16):
                    ntv = jnp.full((16,), i // 8, jnp.int32)
                    nl = lane + ((i % 8) * 16)
                    jv = plsc.load_gather(idx_v, [kv, ntv, tv, nl])
                    qtv = lax.shift_right_logical(jv, 7)
                    jl = jnp.bitwise_and(jv, 127)
                    gv = plsc.load_gather(rows_v, [kv, qtv, tv, jl])
                    new.append(accs[i] + gv)
                return tuple(new)

            accs = lax.fori_loop(
                0, ppw, tbody,
                tuple(jnp.zeros((16,), jnp.float32) for _ in range(NEG // 16)))
            for i in range(NEG // 16):
                acc_v[i // 8, pl.ds((i % 8) * 16, 16)] = accs[i]
            pltpu.sync_copy(acc_v, out_hbm.at[b, pl.ds(wid * NJ, NJ)])

    return sc_gather


def _run_prep(z1, v2, rgb, ni, interpret=False):
    return pl.pallas_call(
        _prep_body,
        grid=(B,),
        in_specs=[
            pl.BlockSpec((1, C, HW), lambda b: (b, 0, 0)),
            pl.BlockSpec((1, C, HW), lambda b: (b, 0, 0)),
            pl.BlockSpec((3, HW), lambda b: (0, 0)),
            pl.BlockSpec((1, 2, HW, NEG), lambda b: (b, 0, 0, 0)),
        ],
        out_specs=[
            pl.BlockSpec((1, NQ * HW, 128), lambda b: (b, 0, 0)),
            pl.BlockSpec((1, NJ * HW, 128), lambda b: (b, 0, 0)),
            pl.BlockSpec((1, 1, 128), lambda b: (b, 0, 0)),
        ],
        out_shape=[
            jax.ShapeDtypeStruct((B, NQ * HW, 128), jnp.float32),
            jax.ShapeDtypeStruct((B, NJ * HW, 128), jnp.int32),
            jax.ShapeDtypeStruct((B, 1, 128), jnp.float32),
        ],
        scratch_shapes=[pltpu.VMEM((HW, HW), jnp.float32)],
        interpret=interpret,
    )(z1, v2, rgb, ni)


def _run_finish(partials, s0, nw, interpret=False):
    return pl.pallas_call(
        _finish_body,
        in_specs=[
            pl.BlockSpec((B, NJ * nw, 128), lambda: (0, 0, 0)),
            pl.BlockSpec((B, 1, 128), lambda: (0, 0, 0)),
        ],
        out_specs=pl.BlockSpec((8, 128), lambda: (0, 0)),
        out_shape=jax.ShapeDtypeStruct((8, 128), jnp.float32),
        interpret=interpret,
    )(partials, s0)


@jax.jit
def kernel(views_1, views_2, img, neg_idx):
    z1 = views_1.reshape(B, C, HW)
    v2 = views_2.reshape(B, C, HW)
    rgb = img[0].reshape(3, HW)
    a_mat, jflat, s0 = _run_prep(z1, v2, rgb, neg_idx)

    info = plsc.get_sparse_core_info()
    nw = info.num_cores * info.num_subcores
    ppw = HW // nw
    partials = _make_sc_gather(info.num_cores, nw, ppw)(a_mat, jflat)

    res = _run_finish(partials, s0, nw)
    return res[0, 0], res[0, 1], res[0, 2]


# speedup vs baseline: 1.8314x; 1.1204x over previous
import functools

import jax
import jax.numpy as jnp
from jax import lax
from jax.experimental import pallas as pl
from jax.experimental.pallas import tpu as pltpu
from jax.experimental.pallas import tpu_sc as plsc

TEMPERATURE = 2.0
FACTOR = 0.8
NEG = 256
EPS = 1e-08

B = 4
C = 192
H = 24
W = 24
HW = H * W
NQ = 5
NJ = NEG // 128


def _prep_body(z_ref, v_ref, rgb_ref, ni_ref, a_ref, j_ref, s0_ref, w_scr):
    b = pl.program_id(0)

    @pl.when(b == 0)
    def _():
        p1 = lax.iota(jnp.int32, HW)
        rows = (p1 // W).astype(jnp.float32)
        cols = (p1 % W).astype(jnp.float32)
        dr = (lax.broadcast_in_dim(rows, (HW, HW), (0,))
              - lax.broadcast_in_dim(rows, (HW, HW), (1,)))
        dc = (lax.broadcast_in_dim(cols, (HW, HW), (0,))
              - lax.broadcast_in_dim(cols, (HW, HW), (1,)))
        diag = float(((H - 1) ** 2 + (W - 1) ** 2) ** 0.5)
        deuc = jnp.sqrt(dr * dr + dc * dc) * (1.0 / diag)
        acc = jnp.zeros((HW, HW), jnp.float32)
        for k in range(3):
            rk = rgb_ref[k]
            rp = lax.broadcast_in_dim(rk, (HW, HW), (0,))
            rq = lax.broadcast_in_dim(rk, (HW, HW), (1,))
            acc = acc + (rp - rq) * (rp - rq)
        drgb = jnp.sqrt(acc) * (1.0 / (3.0 ** 0.5))
        w_scr[...] = deuc * FACTOR + drgb * (1.0 - FACTOR)

    z = z_ref[0]
    v = v_ref[0]
    g = lax.dot_general(z, v, (((1,), (1,)), ((), ())),
                        preferred_element_type=jnp.float32)
    n1sq = jnp.sum(z * z, axis=1)
    n2sq = jnp.sum(v * v, axis=1)
    n1 = jnp.sqrt(n1sq)
    n2 = jnp.sqrt(n2sq)
    n1m = lax.broadcast_in_dim(n1, (HW, HW), (0,))
    n2m = lax.broadcast_in_dim(n2, (HW, HW), (1,))
    r1m = lax.broadcast_in_dim(1.0 / n1, (HW, HW), (0,))
    r2m = lax.broadcast_in_dim(1.0 / n2, (HW, HW), (1,))
    t = jnp.abs(g) * w_scr[...]
    a = jnp.where(n1m * n2m < EPS, t * (1.0 / EPS), t * r1m * r2m)
    a = jnp.minimum(a, 1.0)
    for qt in range(NQ):
        wdt = min(128, HW - qt * 128)
        a_ref[0, pl.ds(qt * HW, HW), pl.ds(0, wdt)] = lax.slice(
            a, (0, qt * 128), (HW, qt * 128 + wdt))
    jf = ni_ref[0, 0] * W + ni_ref[0, 1]
    for nt in range(NJ):
        j_ref[0, pl.ds(nt * HW, HW), :] = lax.slice(
            jf, (0, nt * 128), (HW, nt * 128 + 128))
    s0pix = jnp.minimum(n1sq / jnp.maximum(n1sq, EPS), 1.0)
    s0_ref[...] = jnp.full((1, 1, 128), jnp.sum(s0pix) * (1.0 / HW),
                           jnp.float32)


def _finish_body(p_ref, s0_ref, o_ref):
    parts = p_ref[...]
    ns = [jnp.zeros((B, 128), jnp.float32) for _ in range(NJ)]
    for wkr in range(parts.shape[1] // NJ):
        for nt in range(NJ):
            ns[nt] = ns[nt] + parts[:, NJ * wkr + nt, :]
    sims = [x * (1.0 / (HW * TEMPERATURE)) for x in ns]
    s0 = s0_ref[...][:, 0, 0:1]
    logp0 = jnp.clip(jnp.log(s0), -100.0, None)
    l1m = sum(jnp.sum(jnp.clip(jnp.log(1.0 - s), -100.0, None),
                      axis=1, keepdims=True) for s in sims)
    bce = -(logp0 + l1m) * (1.0 / (NEG + 1))
    loss = jnp.sum(bce) * (1.0 / B)
    out2 = jnp.sum(s0) * (1.0 / B)
    out3 = sum(jnp.sum(s) for s in sims) * (TEMPERATURE / (NEG * B))
    lanes = lax.broadcasted_iota(jnp.int32, (8, 128), 1)
    res = jnp.where(lanes == 0, loss,
                    jnp.where(lanes == 1, out2,
                              jnp.where(lanes == 2, out3, 0.0)))
    o_ref[...] = res


def _make_sc_gather(nc, nw, ppw):
    mesh = plsc.VectorSubcoreMesh(core_axis_name="c", subcore_axis_name="s")

    @functools.partial(
        pl.kernel,
        out_type=jax.ShapeDtypeStruct((B, NJ * nw, 128), jnp.float32),
        mesh=mesh,
        scratch_types=[
            pltpu.VMEM((2, NQ, ppw, 128), jnp.float32),
            pltpu.VMEM((2, NJ, ppw, 128), jnp.int32),
            pltpu.VMEM((NJ, 128), jnp.float32),
            pltpu.SemaphoreType.DMA,
        ],
        compiler_params=pltpu.CompilerParams(use_tc_tiling_on_sc=False,
                                             needs_layout_passes=False),
    )
    def sc_gather(a_hbm, j_hbm, out_hbm, rows_v, idx_v, acc_v, sem):
        cid = lax.axis_index("c")
        sid = lax.axis_index("s")
        wid = sid * nc + cid
        base = wid * ppw
        lane = lax.iota(jnp.int32, 16)

        def start(b, k):
            hs = []
            for qt in range(NQ):
                hs.append(pltpu.async_copy(
                    a_hbm.at[b, pl.ds(qt * HW + base, ppw)],
                    rows_v.at[k, qt], sem))
            for nt in range(NJ):
                hs.append(pltpu.async_copy(
                    j_hbm.at[b, pl.ds(nt * HW + base, ppw)],
                    idx_v.at[k, nt], sem))
            return hs

        pend = start(0, 0)
        for b in range(B):
            k = b % 2
            kv = jnp.full((16,), k, jnp.int32)
            for h in pend:
                h.wait()
            if b + 1 < B:
                pend = start(b + 1, 1 - k)

            def tbody(t, accs):
                tv = jnp.full((16,), t, jnp.int32)
                new = []
                for i in range(NEG // 16):
                    ntv = jnp.full((16,), i // 8, jnp.int32)
                    nl = lane + ((i % 8) * 16)
                    jv = plsc.load_gather(idx_v, [kv, ntv, tv, nl])
                    qtv = lax.shift_right_logical(jv, 7)
                    jl = jnp.bitwise_and(jv, 127)
                    gv = plsc.load_gather(rows_v, [kv, qtv, tv, jl])
                    new.append(accs[i] + gv)
                return tuple(new)

            accs = lax.fori_loop(
                0, ppw, tbody,
                tuple(jnp.zeros((16,), jnp.float32) for _ in range(NEG // 16)))
            for i in range(NEG // 16):
                acc_v[i // 8, pl.ds((i % 8) * 16, 16)] = accs[i]
            pltpu.sync_copy(acc_v, out_hbm.at[b, pl.ds(wid * NJ, NJ)])

    return sc_gather


def _run_prep(z1, v2, rgb, ni, interpret=False):
    return pl.pallas_call(
        _prep_body,
        grid=(B,),
        in_specs=[
            pl.BlockSpec((1, HW, C), lambda b: (b, 0, 0)),
            pl.BlockSpec((1, HW, C), lambda b: (b, 0, 0)),
            pl.BlockSpec((3, HW), lambda b: (0, 0)),
            pl.BlockSpec((1, 2, HW, NEG), lambda b: (b, 0, 0, 0)),
        ],
        out_specs=[
            pl.BlockSpec((1, NQ * HW, 128), lambda b: (b, 0, 0)),
            pl.BlockSpec((1, NJ * HW, 128), lambda b: (b, 0, 0)),
            pl.BlockSpec((1, 1, 128), lambda b: (b, 0, 0)),
        ],
        out_shape=[
            jax.ShapeDtypeStruct((B, NQ * HW, 128), jnp.float32),
            jax.ShapeDtypeStruct((B, NJ * HW, 128), jnp.int32),
            jax.ShapeDtypeStruct((B, 1, 128), jnp.float32),
        ],
        scratch_shapes=[pltpu.VMEM((HW, HW), jnp.float32)],
        interpret=interpret,
    )(z1, v2, rgb, ni)


def _run_finish(partials, s0, nw, interpret=False):
    return pl.pallas_call(
        _finish_body,
        in_specs=[
            pl.BlockSpec((B, NJ * nw, 128), lambda: (0, 0, 0)),
            pl.BlockSpec((B, 1, 128), lambda: (0, 0, 0)),
        ],
        out_specs=pl.BlockSpec((8, 128), lambda: (0, 0)),
        out_shape=jax.ShapeDtypeStruct((8, 128), jnp.float32),
        interpret=interpret,
    )(partials, s0)


@jax.jit
def kernel(views_1, views_2, img, neg_idx):
    z1t = views_1.reshape(B, C, HW).transpose(0, 2, 1)
    v2t = views_2.reshape(B, C, HW).transpose(0, 2, 1)
    rgb = img[0].reshape(3, HW)
    a_mat, jflat, s0 = _run_prep(z1t, v2t, rgb, neg_idx)

    info = plsc.get_sparse_core_info()
    nw = info.num_cores * info.num_subcores
    ppw = HW // nw
    partials = _make_sc_gather(info.num_cores, nw, ppw)(a_mat, jflat)

    res = _run_finish(partials, s0, nw)
    return res[0, 0], res[0, 1], res[0, 2]


# speedup vs baseline: 2.0507x; 1.1198x over previous
import functools

import jax
import jax.numpy as jnp
from jax import lax
from jax.experimental import pallas as pl
from jax.experimental.pallas import tpu as pltpu
from jax.experimental.pallas import tpu_sc as plsc

TEMPERATURE = 2.0
FACTOR = 0.8
NEG = 256
EPS = 1e-08

B = 4
C = 192
H = 24
W = 24
HW = H * W
NQ = 5
NJ = NEG // 128


def _prep_body(z_ref, v_ref, rgb_ref, ni_ref, a_ref, j_ref, s0_ref, w_scr):
    b = pl.program_id(0)

    @pl.when(b == 0)
    def _():
        p1 = lax.iota(jnp.int32, HW)
        rows = (p1 // W).astype(jnp.float32)
        cols = (p1 % W).astype(jnp.float32)
        dr = (lax.broadcast_in_dim(rows, (HW, HW), (0,))
              - lax.broadcast_in_dim(rows, (HW, HW), (1,)))
        dc = (lax.broadcast_in_dim(cols, (HW, HW), (0,))
              - lax.broadcast_in_dim(cols, (HW, HW), (1,)))
        diag = float(((H - 1) ** 2 + (W - 1) ** 2) ** 0.5)
        deuc = jnp.sqrt(dr * dr + dc * dc) * (1.0 / diag)
        acc = jnp.zeros((HW, HW), jnp.float32)
        for k in range(3):
            rk = rgb_ref[k]
            rp = lax.broadcast_in_dim(rk, (HW, HW), (0,))
            rq = lax.broadcast_in_dim(rk, (HW, HW), (1,))
            acc = acc + (rp - rq) * (rp - rq)
        drgb = jnp.sqrt(acc) * (1.0 / (3.0 ** 0.5))
        w_scr[...] = deuc * FACTOR + drgb * (1.0 - FACTOR)

    z = z_ref[0]
    v = v_ref[0]
    g = lax.dot_general(z, v, (((1,), (1,)), ((), ())),
                        preferred_element_type=jnp.float32)
    n1sq = jnp.sum(z * z, axis=1)
    n2sq = jnp.sum(v * v, axis=1)
    n1 = jnp.sqrt(n1sq)
    n2 = jnp.sqrt(n2sq)
    n1m = lax.broadcast_in_dim(n1, (HW, HW), (0,))
    n2m = lax.broadcast_in_dim(n2, (HW, HW), (1,))
    r1m = lax.broadcast_in_dim(1.0 / n1, (HW, HW), (0,))
    r2m = lax.broadcast_in_dim(1.0 / n2, (HW, HW), (1,))
    t = jnp.abs(g) * w_scr[...]
    a = jnp.where(n1m * n2m < EPS, t * (1.0 / EPS), t * r1m * r2m)
    a = jnp.minimum(a, 1.0)
    ap = pltpu.bitcast(a.astype(jnp.bfloat16), jnp.int32)
    for qt in range(NQ):
        wdt = min(128, HW - qt * 128)
        a_ref[0, pl.ds(qt * (HW // 2), HW // 2), pl.ds(0, wdt)] = lax.slice(
            ap, (0, qt * 128), (HW // 2, qt * 128 + wdt))
    jf = ni_ref[0, 0] * W + ni_ref[0, 1]
    jp = pltpu.bitcast(jf.astype(jnp.int16), jnp.int32)
    for nt in range(NJ):
        j_ref[0, pl.ds(nt * (HW // 2), HW // 2), :] = lax.slice(
            jp, (0, nt * 128), (HW // 2, nt * 128 + 128))
    s0pix = jnp.minimum(n1sq / jnp.maximum(n1sq, EPS), 1.0)
    s0_ref[...] = jnp.full((1, 1, 128), jnp.sum(s0pix) * (1.0 / HW),
                           jnp.float32)


def _finish_body(p_ref, s0_ref, o1_ref, o2_ref, o3_ref):
    parts = p_ref[...]
    ns = [jnp.zeros((B, 128), jnp.float32) for _ in range(NJ)]
    for wkr in range(parts.shape[1] // NJ):
        for nt in range(NJ):
            ns[nt] = ns[nt] + parts[:, NJ * wkr + nt, :]
    sims = [x * (1.0 / (HW * TEMPERATURE)) for x in ns]
    s0 = s0_ref[...][:, 0, 0:1]
    logp0 = jnp.clip(jnp.log(s0), -100.0, None)
    l1m = sum(jnp.sum(jnp.clip(jnp.log(1.0 - s), -100.0, None),
                      axis=1, keepdims=True) for s in sims)
    bce = -(logp0 + l1m) * (1.0 / (NEG + 1))
    o1_ref[0, 0] = jnp.sum(bce) * (1.0 / B)
    o2_ref[0, 0] = jnp.sum(s0) * (1.0 / B)
    o3_ref[0, 0] = sum(jnp.sum(s) for s in sims) * (TEMPERATURE / (NEG * B))


def _make_sc_gather(nc, nw, ppw):
    mesh = plsc.VectorSubcoreMesh(core_axis_name="c", subcore_axis_name="s")

    @functools.partial(
        pl.kernel,
        out_type=jax.ShapeDtypeStruct((B, NJ * nw, 128), jnp.float32),
        mesh=mesh,
        scratch_types=[
            pltpu.VMEM((2, NQ, ppw // 2, 128), jnp.int32),
            pltpu.VMEM((2, NJ, ppw // 2, 128), jnp.int32),
            pltpu.VMEM((NJ, 128), jnp.float32),
            pltpu.SemaphoreType.DMA,
        ],
        compiler_params=pltpu.CompilerParams(use_tc_tiling_on_sc=False,
                                             needs_layout_passes=False),
    )
    def sc_gather(a_hbm, j_hbm, out_hbm, rows_v, idx_v, acc_v, sem):
        cid = lax.axis_index("c")
        sid = lax.axis_index("s")
        wid = sid * nc + cid
        ppw2 = ppw // 2
        base2 = wid * ppw2
        lane = lax.iota(jnp.int32, 16)

        def copies(b, k):
            out = []
            for qt in range(NQ):
                out.append((a_hbm.at[b, pl.ds(qt * (HW // 2) + base2, ppw2)],
                            rows_v.at[k, qt]))
            for nt in range(NJ):
                out.append((j_hbm.at[b, pl.ds(nt * (HW // 2) + base2, ppw2)],
                            idx_v.at[k, nt]))
            return out

        def issue(b, k):
            for src, dst in copies(b, k):
                pltpu.async_copy(src, dst, sem)

        def drain(b, k):
            for src, dst in copies(b, k):
                pltpu.make_async_copy(src, dst, sem).wait()

        issue(0, 0)

        @pl.loop(0, B)
        def _(b):
            k = jnp.bitwise_and(b, 1)
            kv = jnp.full((16,), k, jnp.int32)
            drain(b, k)

            @pl.when(b + 1 < B)
            def _():
                issue(b + 1, 1 - k)

            def tbody(t, accs):
                tv = jnp.full((16,), lax.shift_right_logical(t, 1), jnp.int32)
                shv = jnp.full((16,), jnp.bitwise_and(t, 1) * 16, jnp.int32)
                new = []
                for i in range(NEG // 16):
                    ntv = jnp.full((16,), i // 8, jnp.int32)
                    nl = lane + ((i % 8) * 16)
                    jw = plsc.load_gather(idx_v, [kv, ntv, tv, nl])
                    jv = jnp.bitwise_and(lax.shift_right_logical(jw, shv),
                                         0xFFFF)
                    qtv = lax.shift_right_logical(jv, 7)
                    jl = jnp.bitwise_and(jv, 127)
                    gw = plsc.load_gather(rows_v, [kv, qtv, tv, jl])
                    gb = lax.shift_left(lax.shift_right_logical(gw, shv), 16)
                    new.append(accs[i] + plsc.bitcast(gb, jnp.float32))
                return tuple(new)

            accs = lax.fori_loop(
                0, ppw, tbody,
                tuple(jnp.zeros((16,), jnp.float32) for _ in range(NEG // 16)))
            for i in range(NEG // 16):
                acc_v[i // 8, pl.ds((i % 8) * 16, 16)] = accs[i]
            pltpu.sync_copy(acc_v, out_hbm.at[b, pl.ds(wid * NJ, NJ)])

    return sc_gather


def _run_prep(z1, v2, rgb, ni, interpret=False):
    return pl.pallas_call(
        _prep_body,
        grid=(B,),
        in_specs=[
            pl.BlockSpec((1, HW, C), lambda b: (b, 0, 0)),
            pl.BlockSpec((1, HW, C), lambda b: (b, 0, 0)),
            pl.BlockSpec((3, HW), lambda b: (0, 0)),
            pl.BlockSpec((1, 2, HW, NEG), lambda b: (b, 0, 0, 0)),
        ],
        out_specs=[
            pl.BlockSpec((1, NQ * (HW // 2), 128), lambda b: (b, 0, 0)),
            pl.BlockSpec((1, NJ * (HW // 2), 128), lambda b: (b, 0, 0)),
            pl.BlockSpec((1, 1, 128), lambda b: (b, 0, 0)),
        ],
        out_shape=[
            jax.ShapeDtypeStruct((B, NQ * (HW // 2), 128), jnp.int32),
            jax.ShapeDtypeStruct((B, NJ * (HW // 2), 128), jnp.int32),
            jax.ShapeDtypeStruct((B, 1, 128), jnp.float32),
        ],
        scratch_shapes=[pltpu.VMEM((HW, HW), jnp.float32)],
        interpret=interpret,
    )(z1, v2, rgb, ni)


def _run_finish(partials, s0, nw, interpret=False):
    return pl.pallas_call(
        _finish_body,
        in_specs=[
            pl.BlockSpec((B, NJ * nw, 128), lambda: (0, 0, 0)),
            pl.BlockSpec((B, 1, 128), lambda: (0, 0, 0)),
        ],
        out_specs=[pl.BlockSpec(memory_space=pltpu.SMEM)] * 3,
        out_shape=[jax.ShapeDtypeStruct((1, 1), jnp.float32)] * 3,
        interpret=interpret,
    )(partials, s0)


@jax.jit
def kernel(views_1, views_2, img, neg_idx):
    z1t = views_1.reshape(B, C, HW).transpose(0, 2, 1)
    v2t = views_2.reshape(B, C, HW).transpose(0, 2, 1)
    rgb = img[0].reshape(3, HW)
    a_mat, jflat, s0 = _run_prep(z1t, v2t, rgb, neg_idx)

    info = plsc.get_sparse_core_info()
    nw = info.num_cores * info.num_subcores
    ppw = HW // nw
    partials = _make_sc_gather(info.num_cores, nw, ppw)(a_mat, jflat)

    r1, r2, r3 = _run_finish(partials, s0, nw)
    return r1[0, 0], r2[0, 0], r3[0, 0]
